# baseline (speedup 1.0000x reference)
import functools

import jax
import jax.numpy as jnp
from jax import lax
from jax.experimental import pallas as pl
from jax.experimental.pallas import tpu as pltpu
from jax.experimental.pallas import tpu_sc as plsc

_D = 26
_L = 100
_H = 2048
_M = 4
_K = 4
_DPAD = 28
_NT = _DPAD // _K


def _sc_encode(x_sc, flat_tab, n_sc):
    info = plsc.get_sparse_core_info()
    nc, ns = info.num_cores, info.num_subcores
    nw = nc * ns
    n_b = n_sc // nw
    n_grp = n_b // _M

    xp = jnp.pad(x_sc, ((0, 0), (0, _DPAD - _D)))
    xp = xp.reshape(nw, n_grp, _M, _NT, _K).transpose(0, 1, 3, 2, 4)
    xg = xp.reshape(nw * n_grp, _NT * 16)

    mesh = plsc.VectorSubcoreMesh(core_axis_name="c", subcore_axis_name="s")

    @functools.partial(
        pl.kernel,
        mesh=mesh,
        out_type=jax.ShapeDtypeStruct((n_sc, _H), jnp.float32),
        scratch_types=[
            pltpu.VMEM((_NT * 16,), jnp.float32),
            pltpu.VMEM((16, _H), jnp.float32),
            pltpu.VMEM((16, _H), jnp.float32),
            pltpu.VMEM((_M, _H), jnp.float32),
            pltpu.SemaphoreType.DMA,
        ],
    )
    def k(xt_hbm, tab_hbm, out_hbm, xt_v, rlo_v, rhi_v, acc_v, sem):
        wid = lax.axis_index("s") * nc + lax.axis_index("c")
        lane = lax.broadcasted_iota(jnp.int32, (16,), 0)
        lane_j = lane & (_K - 1)
        zero16f = jnp.zeros((16,), jnp.float32)

        def grp_body(grp, carry):
            pltpu.sync_copy(xt_hbm.at[wid * n_grp + grp], xt_v)

            for b in range(_M):
                def z_body(j, c, b=b):
                    acc_v[b, pl.ds(pl.multiple_of(j * 16, 16), 16)] = zero16f
                    return c
                lax.fori_loop(0, _H // 16, z_body, 0)

            def t_body(t, carry2):
                xv = xt_v[pl.ds(pl.multiple_of(t * 16, 16), 16)]
                xn = jnp.clip(xv * float(_L - 1), 0.0, float(_L - 1))
                low_i = xn.astype(jnp.int32)
                a = xn - low_i.astype(jnp.float32)
                high_i = jnp.minimum(low_i + 1, _L - 1)
                featv = lane_j + t * _K
                valid = featv < _D
                offv = featv * _L
                il = jnp.where(valid, low_i + offv, 0)
                ih = jnp.where(valid, high_i + offv, 0)
                wl = jnp.where(valid, 1.0 - a, zero16f)
                wh = jnp.where(valid, a, zero16f)
                cl = pltpu.async_copy(tab_hbm.at[il], rlo_v, sem)
                ch = pltpu.async_copy(tab_hbm.at[ih], rhi_v, sem)
                cl.wait()
                ch.wait()
                for b in range(_M):
                    wls = [wl[b * _K + j] for j in range(_K)]
                    whs = [wh[b * _K + j] for j in range(_K)]

                    def col_body(j, c, b=b, wls=wls, whs=whs):
                        sl = pl.ds(pl.multiple_of(j * 16, 16), 16)
                        v = acc_v[b, sl]
                        for r in range(_K):
                            v = v + wls[r] * rlo_v[b * _K + r, sl]
                            v = v + whs[r] * rhi_v[b * _K + r, sl]
                        acc_v[b, sl] = v
                        return c

                    lax.fori_loop(0, _H // 16, col_body, 0)
                return carry2

            lax.fori_loop(0, _NT, t_body, 0)

            for b in range(_M):
                def ss_body(j, ssv, b=b):
                    v = acc_v[b, pl.ds(pl.multiple_of(j * 16, 16), 16)]
                    return ssv + v * v

                ssv = lax.fori_loop(0, _H // 16, ss_body, zero16f)
                dn = lax.GatherDimensionNumbers(
                    offset_dims=(), collapsed_slice_dims=(0,),
                    start_index_map=(0,))
                ss = ssv
                for s in (1, 2, 4, 8):
                    perm = (lane ^ s)[:, None]
                    ss = ss + lax.gather(
                        ss, perm, dn, slice_sizes=(1,),
                        mode=lax.GatherScatterMode.PROMISE_IN_BOUNDS)
                ib = lax.bitcast_convert_type(ss, jnp.int32)
                ih2 = jnp.int32(0x5F3759DF) - lax.shift_right_arithmetic(ib, 1)
                y = lax.bitcast_convert_type(ih2, jnp.float32)
                half = ss * 0.5
                for _ in range(3):
                    y = y * (1.5 - half * y * y)

                def sc_body(j, c, b=b, y=y):
                    sl = pl.ds(pl.multiple_of(j * 16, 16), 16)
                    acc_v[b, sl] = acc_v[b, sl] * y
                    return c

                lax.fori_loop(0, _H // 16, sc_body, 0)

            pltpu.sync_copy(acc_v,
                            out_hbm.at[pl.ds(wid * n_b + grp * _M, _M)])
            return carry

        lax.fori_loop(0, n_grp, grp_body, 0)

    return k(xg, flat_tab)


def kernel(x, base_hvs):
    if x.ndim == 1:
        x = x[None, :]
    B, D = x.shape
    _, L, H = base_hvs.shape
    flat = base_hvs.reshape(D * L, H)
    return _sc_encode(x, flat, B)


# speedup vs baseline: 4.7665x; 4.7665x over previous
import functools

import jax
import jax.numpy as jnp
from jax import lax
from jax.experimental import pallas as pl
from jax.experimental.pallas import tpu as pltpu
from jax.experimental.pallas import tpu_sc as plsc

_D = 26
_L = 100
_H = 2048
_M = 4
_K = 4
_DPAD = 28
_NT = _DPAD // _K


def _sc_encode(x_sc, flat_tab, n_sc):
    info = plsc.get_sparse_core_info()
    nc, ns = info.num_cores, info.num_subcores
    nw = nc * ns
    n_b = n_sc // nw
    n_grp = n_b // _M

    xp = jnp.pad(x_sc, ((0, 0), (0, _DPAD - _D)))
    xp = xp.reshape(nw, n_grp, _M, _NT, _K).transpose(0, 1, 3, 2, 4)
    xg = xp.reshape(nw * n_grp, _NT * 16)

    mesh = plsc.VectorSubcoreMesh(core_axis_name="c", subcore_axis_name="s")

    @functools.partial(
        pl.kernel,
        mesh=mesh,
        out_type=jax.ShapeDtypeStruct((n_sc, _H), jnp.float32),
        scratch_types=[
            pltpu.VMEM((_NT * 16,), jnp.float32),
            pltpu.VMEM((16, _H), jnp.float32),
            pltpu.VMEM((16, _H), jnp.float32),
            pltpu.VMEM((_M, _H), jnp.float32),
            pltpu.SemaphoreType.DMA,
        ],
    )
    def k(xt_hbm, tab_hbm, out_hbm, xt_v, rlo_v, rhi_v, acc_v, sem):
        wid = lax.axis_index("s") * nc + lax.axis_index("c")
        lane = lax.broadcasted_iota(jnp.int32, (16,), 0)
        lane_j = lane & (_K - 1)
        zero16f = jnp.zeros((16,), jnp.float32)

        def grp_body(grp, carry):
            pltpu.sync_copy(xt_hbm.at[wid * n_grp + grp], xt_v)

            for b in range(_M):
                def z_body(j, c, b=b):
                    acc_v[b, pl.ds(pl.multiple_of(j * 16, 16), 16)] = zero16f
                    return c
                lax.fori_loop(0, _H // 16, z_body, 0)

            def t_body(t, carry2):
                xv = xt_v[pl.ds(pl.multiple_of(t * 16, 16), 16)]
                xn = jnp.clip(xv * float(_L - 1), 0.0, float(_L - 1))
                low_i = xn.astype(jnp.int32)
                a = xn - low_i.astype(jnp.float32)
                high_i = jnp.minimum(low_i + 1, _L - 1)
                featv = lane_j + t * _K
                valid = featv < _D
                offv = featv * _L
                il = jnp.where(valid, low_i + offv, 0)
                ih = jnp.where(valid, high_i + offv, 0)
                wl = jnp.where(valid, 1.0 - a, zero16f)
                wh = jnp.where(valid, a, zero16f)
                cl = pltpu.async_copy(tab_hbm.at[il], rlo_v, sem)
                ch = pltpu.async_copy(tab_hbm.at[ih], rhi_v, sem)
                cl.wait()
                ch.wait()
                for b in range(_M):
                    wls = [wl[b * _K + j] for j in range(_K)]
                    whs = [wh[b * _K + j] for j in range(_K)]

                    def col_body(j, c, b=b, wls=wls, whs=whs):
                        sl = pl.ds(pl.multiple_of(j * 16, 16), 16)
                        v = acc_v[b, sl]
                        for r in range(_K):
                            v = v + wls[r] * rlo_v[b * _K + r, sl]
                            v = v + whs[r] * rhi_v[b * _K + r, sl]
                        acc_v[b, sl] = v
                        return c

                    lax.fori_loop(0, _H // 16, col_body, 0)
                return carry2

            lax.fori_loop(0, _NT, t_body, 0)

            for b in range(_M):
                def ss_body(j, ssv, b=b):
                    v = acc_v[b, pl.ds(pl.multiple_of(j * 16, 16), 16)]
                    return ssv + v * v

                ssv = lax.fori_loop(0, _H // 16, ss_body, zero16f)
                dn = lax.GatherDimensionNumbers(
                    offset_dims=(), collapsed_slice_dims=(0,),
                    start_index_map=(0,))
                ss = ssv
                for s in (1, 2, 4, 8):
                    perm = (lane ^ s)[:, None]
                    ss = ss + lax.gather(
                        ss, perm, dn, slice_sizes=(1,),
                        mode=lax.GatherScatterMode.PROMISE_IN_BOUNDS)
                ib = lax.bitcast_convert_type(ss, jnp.int32)
                ih2 = jnp.int32(0x5F3759DF) - lax.shift_right_arithmetic(ib, 1)
                y = lax.bitcast_convert_type(ih2, jnp.float32)
                half = ss * 0.5
                for _ in range(3):
                    y = y * (1.5 - half * y * y)

                def sc_body(j, c, b=b, y=y):
                    sl = pl.ds(pl.multiple_of(j * 16, 16), 16)
                    acc_v[b, sl] = acc_v[b, sl] * y
                    return c

                lax.fori_loop(0, _H // 16, sc_body, 0)

            pltpu.sync_copy(acc_v,
                            out_hbm.at[pl.ds(wid * n_b + grp * _M, _M)])
            return carry

        lax.fori_loop(0, n_grp, grp_body, 0)

    return k(xg, flat_tab)


def _tc_body(x_ref, tab_ref, o_ref):
    x = x_ref[...]
    Bt, D = x.shape
    H = tab_ref.shape[-1]
    xn = jnp.clip(x * float(_L - 1), 0.0, float(_L - 1))
    low_f = jnp.floor(xn)
    a = xn - low_f
    low_i = low_f.astype(jnp.int32)
    high_i = jnp.minimum(low_i + 1, _L - 1)
    lvl = jax.lax.broadcasted_iota(jnp.int32, (Bt, _L), 1)
    acc = jnp.zeros((Bt, H), jnp.float32)
    for d in range(D):
        w = jnp.where(lvl == low_i[:, d:d + 1], 1.0 - a[:, d:d + 1], 0.0)
        w = w + jnp.where(lvl == high_i[:, d:d + 1], a[:, d:d + 1], 0.0)
        acc = acc + jnp.dot(w, tab_ref[d], preferred_element_type=jnp.float32)
    ss = jnp.sum(acc * acc, axis=1, keepdims=True)
    o_ref[...] = acc / jnp.sqrt(ss)


def _tc_encode(x_tc, base_hvs, bt):
    B, D = x_tc.shape
    _, L, H = base_hvs.shape
    return pl.pallas_call(
        _tc_body,
        grid=(B // bt,),
        in_specs=[
            pl.BlockSpec((bt, D), lambda i: (i, 0)),
            pl.BlockSpec((D, L, H), lambda i: (0, 0, 0)),
        ],
        out_specs=pl.BlockSpec((bt, H), lambda i: (i, 0)),
        out_shape=jax.ShapeDtypeStruct((B, H), jnp.float32),
    )(x_tc, base_hvs)


_N_SC = 128


def kernel(x, base_hvs):
    if x.ndim == 1:
        x = x[None, :]
    B, D = x.shape
    _, L, H = base_hvs.shape
    flat = base_hvs.reshape(D * L, H)
    n_sc = _N_SC if B % 512 == 0 else 0
    if n_sc == 0:
        return _tc_encode(x, base_hvs, min(B, 128) if B % 128 == 0 else B)
    n_tc = B - n_sc
    out_tc = _tc_encode(x[:n_tc], base_hvs, 128)
    out_sc = _sc_encode(x[n_tc:], flat, n_sc)
    return jnp.concatenate([out_tc, out_sc], axis=0)


# speedup vs baseline: 4.8173x; 1.0107x over previous
import functools

import jax
import jax.numpy as jnp
from jax import lax
from jax.experimental import pallas as pl
from jax.experimental.pallas import tpu as pltpu
from jax.experimental.pallas import tpu_sc as plsc

_D = 26
_L = 100
_H = 2048
_M = 4
_K = 4
_DPAD = 28
_NT = _DPAD // _K


def _sc_encode(x_sc, flat_tab, n_sc):
    info = plsc.get_sparse_core_info()
    nc, ns = info.num_cores, info.num_subcores
    nw = nc * ns
    n_b = n_sc // nw
    n_grp = n_b // _M

    xp = jnp.pad(x_sc, ((0, 0), (0, _DPAD - _D)))
    xp = xp.reshape(nw, n_grp, _M, _NT, _K).transpose(0, 1, 3, 2, 4)
    xg = xp.reshape(nw * n_grp, _NT * 16)

    mesh = plsc.VectorSubcoreMesh(core_axis_name="c", subcore_axis_name="s")

    @functools.partial(
        pl.kernel,
        mesh=mesh,
        out_type=jax.ShapeDtypeStruct((n_sc, _H), jnp.float32),
        scratch_types=[
            pltpu.VMEM((_NT * 16,), jnp.float32),
            pltpu.VMEM((16, _H), jnp.float32),
            pltpu.VMEM((16, _H), jnp.float32),
            pltpu.VMEM((_M, _H), jnp.float32),
            pltpu.SemaphoreType.DMA,
        ],
    )
    def k(xt_hbm, tab_hbm, out_hbm, xt_v, rlo_v, rhi_v, acc_v, sem):
        wid = lax.axis_index("s") * nc + lax.axis_index("c")
        lane = lax.broadcasted_iota(jnp.int32, (16,), 0)
        lane_j = lane & (_K - 1)
        zero16f = jnp.zeros((16,), jnp.float32)

        def grp_body(grp, carry):
            pltpu.sync_copy(xt_hbm.at[wid * n_grp + grp], xt_v)

            for b in range(_M):
                def z_body(j, c, b=b):
                    acc_v[b, pl.ds(pl.multiple_of(j * 16, 16), 16)] = zero16f
                    return c
                lax.fori_loop(0, _H // 16, z_body, 0, unroll=8)

            def t_body(t, carry2):
                xv = xt_v[pl.ds(pl.multiple_of(t * 16, 16), 16)]
                xn = jnp.clip(xv * float(_L - 1), 0.0, float(_L - 1))
                low_i = xn.astype(jnp.int32)
                a = xn - low_i.astype(jnp.float32)
                high_i = jnp.minimum(low_i + 1, _L - 1)
                featv = lane_j + t * _K
                valid = featv < _D
                offv = featv * _L
                il = jnp.where(valid, low_i + offv, 0)
                ih = jnp.where(valid, high_i + offv, 0)
                wl = jnp.where(valid, 1.0 - a, zero16f)
                wh = jnp.where(valid, a, zero16f)
                cl = pltpu.async_copy(tab_hbm.at[il], rlo_v, sem)
                ch = pltpu.async_copy(tab_hbm.at[ih], rhi_v, sem)
                cl.wait()
                ch.wait()
                for b in range(_M):
                    wls = [wl[b * _K + j] for j in range(_K)]
                    whs = [wh[b * _K + j] for j in range(_K)]

                    def col_body(j, c, b=b, wls=wls, whs=whs):
                        sl = pl.ds(pl.multiple_of(j * 16, 16), 16)
                        v = acc_v[b, sl]
                        for r in range(_K):
                            v = v + wls[r] * rlo_v[b * _K + r, sl]
                            v = v + whs[r] * rhi_v[b * _K + r, sl]
                        acc_v[b, sl] = v
                        return c

                    lax.fori_loop(0, _H // 16, col_body, 0, unroll=8)
                return carry2

            lax.fori_loop(0, _NT, t_body, 0)

            for b in range(_M):
                def ss_body(j, ssv, b=b):
                    v = acc_v[b, pl.ds(pl.multiple_of(j * 16, 16), 16)]
                    return ssv + v * v

                ssv = lax.fori_loop(0, _H // 16, ss_body, zero16f, unroll=8)
                dn = lax.GatherDimensionNumbers(
                    offset_dims=(), collapsed_slice_dims=(0,),
                    start_index_map=(0,))
                ss = ssv
                for s in (1, 2, 4, 8):
                    perm = (lane ^ s)[:, None]
                    ss = ss + lax.gather(
                        ss, perm, dn, slice_sizes=(1,),
                        mode=lax.GatherScatterMode.PROMISE_IN_BOUNDS)
                ib = lax.bitcast_convert_type(ss, jnp.int32)
                ih2 = jnp.int32(0x5F3759DF) - lax.shift_right_arithmetic(ib, 1)
                y = lax.bitcast_convert_type(ih2, jnp.float32)
                half = ss * 0.5
                for _ in range(3):
                    y = y * (1.5 - half * y * y)

                def sc_body(j, c, b=b, y=y):
                    sl = pl.ds(pl.multiple_of(j * 16, 16), 16)
                    acc_v[b, sl] = acc_v[b, sl] * y
                    return c

                lax.fori_loop(0, _H // 16, sc_body, 0, unroll=8)

            pltpu.sync_copy(acc_v,
                            out_hbm.at[pl.ds(wid * n_b + grp * _M, _M)])
            return carry

        lax.fori_loop(0, n_grp, grp_body, 0)

    return k(xg, flat_tab)


def _tc_body(x_ref, tab_ref, o_ref):
    x = x_ref[...]
    Bt, D = x.shape
    H = tab_ref.shape[-1]
    xn = jnp.clip(x * float(_L - 1), 0.0, float(_L - 1))
    low_f = jnp.floor(xn)
    a = xn - low_f
    low_i = low_f.astype(jnp.int32)
    high_i = jnp.minimum(low_i + 1, _L - 1)
    lvl = jax.lax.broadcasted_iota(jnp.int32, (Bt, _L), 1)
    acc = jnp.zeros((Bt, H), jnp.float32)
    for d in range(D):
        w = jnp.where(lvl == low_i[:, d:d + 1], 1.0 - a[:, d:d + 1], 0.0)
        w = w + jnp.where(lvl == high_i[:, d:d + 1], a[:, d:d + 1], 0.0)
        acc = acc + jnp.dot(w, tab_ref[d], preferred_element_type=jnp.float32)
    ss = jnp.sum(acc * acc, axis=1, keepdims=True)
    o_ref[...] = acc / jnp.sqrt(ss)


def _tc_encode(x_tc, base_hvs, bt):
    B, D = x_tc.shape
    _, L, H = base_hvs.shape
    return pl.pallas_call(
        _tc_body,
        grid=(B // bt,),
        in_specs=[
            pl.BlockSpec((bt, D), lambda i: (i, 0)),
            pl.BlockSpec((D, L, H), lambda i: (0, 0, 0)),
        ],
        out_specs=pl.BlockSpec((bt, H), lambda i: (i, 0)),
        out_shape=jax.ShapeDtypeStruct((B, H), jnp.float32),
    )(x_tc, base_hvs)


_N_SC = 128


def kernel(x, base_hvs):
    if x.ndim == 1:
        x = x[None, :]
    B, D = x.shape
    _, L, H = base_hvs.shape
    flat = base_hvs.reshape(D * L, H)
    n_sc = _N_SC if B % 512 == 0 else 0
    if n_sc == 0:
        return _tc_encode(x, base_hvs, min(B, 128) if B % 128 == 0 else B)
    n_tc = B - n_sc
    out_tc = _tc_encode(x[:n_tc], base_hvs, 128)
    out_sc = _sc_encode(x[n_tc:], flat, n_sc)
    return jnp.concatenate([out_tc, out_sc], axis=0)


# speedup vs baseline: 9.2816x; 1.9267x over previous
import functools

import jax
import jax.numpy as jnp
from jax import lax
from jax.experimental import pallas as pl
from jax.experimental.pallas import tpu as pltpu
from jax.experimental.pallas import tpu_sc as plsc

_D = 26
_L = 100
_H = 2048
_M = 2
_K = 4
_DPAD = 28
_NT = _DPAD // _K


def _sc_encode(x_sc, flat_tab, n_sc):
    info = plsc.get_sparse_core_info()
    nc, ns = info.num_cores, info.num_subcores
    nw = nc * ns
    n_b = n_sc // nw
    n_grp = n_b // _M

    xp = jnp.pad(x_sc, ((0, 0), (0, _DPAD - _D)))
    xp = xp.reshape(nw, n_grp, _M, _NT, _K).transpose(0, 1, 3, 2, 4)
    xp = jnp.broadcast_to(xp[:, :, :, :, None, :],
                          (nw, n_grp, _NT, _M, 2, _K))
    xg = xp.reshape(nw * n_grp, _NT * 16)

    mesh = plsc.VectorSubcoreMesh(core_axis_name="c", subcore_axis_name="s")

    @functools.partial(
        pl.kernel,
        mesh=mesh,
        out_type=jax.ShapeDtypeStruct((n_sc, _H), jnp.float32),
        scratch_types=[
            pltpu.VMEM((_NT * 16,), jnp.float32),
            pltpu.VMEM((16, _H), jnp.float32),
            pltpu.VMEM((16, _H), jnp.float32),
            pltpu.VMEM((_M, _H), jnp.float32),
            pltpu.SemaphoreType.DMA,
            pltpu.SemaphoreType.DMA,
        ],
    )
    def k(xt_hbm, tab_hbm, out_hbm, xt_v, rows_a, rows_b, acc_v, sem_a, sem_b):
        wid = lax.axis_index("s") * nc + lax.axis_index("c")
        lane = lax.broadcasted_iota(jnp.int32, (16,), 0)
        lane_j = lane & (_K - 1)
        lane_s = lax.shift_right_logical(lane, 2) & 1
        zero16f = jnp.zeros((16,), jnp.float32)
        bufs = [(rows_a, sem_a), (rows_b, sem_b)]

        def grp_body(grp, carry):
            pltpu.sync_copy(xt_hbm.at[wid * n_grp + grp], xt_v)

            ils, wss = [], []
            for t in range(_NT):
                xv = xt_v[pl.ds(t * 16, 16)]
                xn = jnp.clip(xv * float(_L - 1), 0.0, float(_L - 1))
                low_i = xn.astype(jnp.int32)
                a = xn - low_i.astype(jnp.float32)
                high_i = jnp.minimum(low_i + 1, _L - 1)
                featv = lane_j + t * _K
                valid = featv < _D
                lvl = jnp.where(lane_s == 0, low_i, high_i)
                w = jnp.where(lane_s == 0, 1.0 - a, a)
                ils.append(jnp.where(valid, lvl + featv * _L, 0))
                wss.append(jnp.where(valid, w, zero16f))

            def gather(t):
                buf, sem = bufs[t % 2]
                return pltpu.async_copy(tab_hbm.at[ils[t]], buf, sem)

            gather(0)
            for t in range(_NT):
                if t + 1 < _NT:
                    gather(t + 1)
                buf, sem = bufs[t % 2]
                pltpu.make_async_copy(tab_hbm.at[ils[t]], buf, sem).wait()
                for b in range(_M):
                    ws = [wss[t][b * 8 + r] for r in range(8)]

                    def col_body(j, c, t=t, b=b, buf=buf, ws=ws):
                        sl = pl.ds(pl.multiple_of(j * 16, 16), 16)
                        v = ws[0] * buf[b * 8, sl]
                        for r in range(1, 8):
                            v = v + ws[r] * buf[b * 8 + r, sl]
                        if t > 0:
                            v = v + acc_v[b, sl]
                        acc_v[b, sl] = v
                        return c

                    lax.fori_loop(0, _H // 16, col_body, 0, unroll=8)

            for b in range(_M):
                def ss_body(j, ssv, b=b):
                    v = acc_v[b, pl.ds(pl.multiple_of(j * 16, 16), 16)]
                    return ssv + v * v

                ssv = lax.fori_loop(0, _H // 16, ss_body, zero16f, unroll=8)
                dn = lax.GatherDimensionNumbers(
                    offset_dims=(), collapsed_slice_dims=(0,),
                    start_index_map=(0,))
                ss = ssv
                for s in (1, 2, 4, 8):
                    perm = (lane ^ s)[:, None]
                    ss = ss + lax.gather(
                        ss, perm, dn, slice_sizes=(1,),
                        mode=lax.GatherScatterMode.PROMISE_IN_BOUNDS)
                ib = lax.bitcast_convert_type(ss, jnp.int32)
                ih2 = jnp.int32(0x5F3759DF) - lax.shift_right_arithmetic(ib, 1)
                y = lax.bitcast_convert_type(ih2, jnp.float32)
                half = ss * 0.5
                for _ in range(3):
                    y = y * (1.5 - half * y * y)

                def nm_body(j, c, b=b, y=y):
                    sl = pl.ds(pl.multiple_of(j * 16, 16), 16)
                    acc_v[b, sl] = acc_v[b, sl] * y
                    return c

                lax.fori_loop(0, _H // 16, nm_body, 0, unroll=8)

            pltpu.sync_copy(acc_v,
                            out_hbm.at[pl.ds(wid * n_b + grp * _M, _M)])
            return carry

        lax.fori_loop(0, n_grp, grp_body, 0)

    return k(xg, flat_tab)


def _tc_body(x_ref, tab_ref, o_ref):
    x = x_ref[...]
    Bt, D = x.shape
    H = tab_ref.shape[-1]
    xn = jnp.clip(x * float(_L - 1), 0.0, float(_L - 1))
    low_f = jnp.floor(xn)
    a = xn - low_f
    low_i = low_f.astype(jnp.int32)
    high_i = jnp.minimum(low_i + 1, _L - 1)
    lvl = jax.lax.broadcasted_iota(jnp.int32, (Bt, _L), 1)
    acc = jnp.zeros((Bt, H), jnp.float32)
    for d in range(D):
        w = jnp.where(lvl == low_i[:, d:d + 1], 1.0 - a[:, d:d + 1], 0.0)
        w = w + jnp.where(lvl == high_i[:, d:d + 1], a[:, d:d + 1], 0.0)
        acc = acc + jnp.dot(w, tab_ref[d], preferred_element_type=jnp.float32)
    ss = jnp.sum(acc * acc, axis=1, keepdims=True)
    o_ref[...] = acc / jnp.sqrt(ss)


def _tc_encode(x_tc, base_hvs, bt, n_rows, out_rows):
    B, D = x_tc.shape
    _, L, H = base_hvs.shape
    return pl.pallas_call(
        _tc_body,
        grid=(n_rows // bt,),
        in_specs=[
            pl.BlockSpec((bt, D), lambda i: (i, 0)),
            pl.BlockSpec((D, L, H), lambda i: (0, 0, 0)),
        ],
        out_specs=pl.BlockSpec((bt, H), lambda i: (i, 0)),
        out_shape=jax.ShapeDtypeStruct((out_rows, H), jnp.float32),
    )(x_tc, base_hvs)


_N_SC = 64


def kernel(x, base_hvs):
    if x.ndim == 1:
        x = x[None, :]
    B, D = x.shape
    _, L, H = base_hvs.shape
    n_sc = _N_SC if B % 512 == 0 else 0
    if n_sc == 0:
        bt = 128 if B % 128 == 0 else B
        return _tc_encode(x, base_hvs, bt, B, B)
    n_tc = B - n_sc
    flat = base_hvs.reshape(D * L, H)
    out_tc = _tc_encode(x, base_hvs, 120, n_tc, B)
    out_sc = _sc_encode(x[n_tc:], flat, n_sc)
    return lax.dynamic_update_slice(out_tc, out_sc, (n_tc, 0))


# speedup vs baseline: 9.2995x; 1.0019x over previous
import functools

import jax
import jax.numpy as jnp
from jax import lax
from jax.experimental import pallas as pl
from jax.experimental.pallas import tpu as pltpu
from jax.experimental.pallas import tpu_sc as plsc

_D = 26
_L = 100
_H = 2048


def _sc_encode(x_sc, flat_tab, n_sc):
    info = plsc.get_sparse_core_info()
    nc, ns = info.num_cores, info.num_subcores
    nw = nc * ns
    n_b = n_sc // nw
    m = min(2, n_b)
    kf = 16 // (2 * m)
    dpad = -(-_D // kf) * kf
    nt = dpad // kf
    n_grp = n_b // m

    xp = jnp.pad(x_sc, ((0, 0), (0, dpad - _D)))
    xp = xp.reshape(nw, n_grp, m, nt, kf).transpose(0, 1, 3, 2, 4)
    xp = jnp.broadcast_to(xp[:, :, :, :, None, :],
                          (nw, n_grp, nt, m, 2, kf))
    xg = xp.reshape(nw * n_grp, nt * 16)

    mesh = plsc.VectorSubcoreMesh(core_axis_name="c", subcore_axis_name="s")

    @functools.partial(
        pl.kernel,
        mesh=mesh,
        out_type=jax.ShapeDtypeStruct((n_sc, _H), jnp.float32),
        scratch_types=[
            pltpu.VMEM((nt * 16,), jnp.float32),
            pltpu.VMEM((16, _H), jnp.float32),
            pltpu.VMEM((16, _H), jnp.float32),
            pltpu.VMEM((max(m, 2), _H), jnp.float32),
            pltpu.SemaphoreType.DMA,
            pltpu.SemaphoreType.DMA,
        ],
    )
    def k(xt_hbm, tab_hbm, out_hbm, xt_v, rows_a, rows_b, acc_v, sem_a, sem_b):
        wid = lax.axis_index("s") * nc + lax.axis_index("c")
        lane = lax.broadcasted_iota(jnp.int32, (16,), 0)
        lane_j = lane & (kf - 1)
        lane_s = lax.shift_right_logical(lane, kf.bit_length() - 1) & 1
        zero16f = jnp.zeros((16,), jnp.float32)
        bufs = [(rows_a, sem_a), (rows_b, sem_b)]

        def grp_body(grp, carry):
            pltpu.sync_copy(xt_hbm.at[wid * n_grp + grp], xt_v)

            ils, wss = [], []
            for t in range(nt):
                xv = xt_v[pl.ds(t * 16, 16)]
                xn = jnp.clip(xv * float(_L - 1), 0.0, float(_L - 1))
                low_i = xn.astype(jnp.int32)
                a = xn - low_i.astype(jnp.float32)
                high_i = jnp.minimum(low_i + 1, _L - 1)
                featv = lane_j + t * kf
                valid = featv < _D
                lvl = jnp.where(lane_s == 0, low_i, high_i)
                w = jnp.where(lane_s == 0, 1.0 - a, a)
                ils.append(jnp.where(valid, lvl + featv * _L, 0))
                wss.append(jnp.where(valid, w, zero16f))

            def gather(t):
                buf, sem = bufs[t % 2]
                return pltpu.async_copy(tab_hbm.at[ils[t]], buf, sem)

            gather(0)
            nr = 2 * kf
            for t in range(nt):
                if t + 1 < nt:
                    gather(t + 1)
                buf, sem = bufs[t % 2]
                pltpu.make_async_copy(tab_hbm.at[ils[t]], buf, sem).wait()
                for b in range(m):
                    ws = [wss[t][b * nr + r] for r in range(nr)]

                    def col_body(j, c, t=t, b=b, buf=buf, ws=ws):
                        sl = pl.ds(pl.multiple_of(j * 16, 16), 16)
                        v = ws[0] * buf[b * nr, sl]
                        for r in range(1, nr):
                            v = v + ws[r] * buf[b * nr + r, sl]
                        if t > 0:
                            v = v + acc_v[b, sl]
                        acc_v[b, sl] = v
                        return c

                    lax.fori_loop(0, _H // 16, col_body, 0,
                                  unroll=(64 // nr))

            for b in range(m):
                def ss_body(j, ssv, b=b):
                    v = acc_v[b, pl.ds(pl.multiple_of(j * 16, 16), 16)]
                    return ssv + v * v

                ssv = lax.fori_loop(0, _H // 16, ss_body, zero16f, unroll=8)
                dn = lax.GatherDimensionNumbers(
                    offset_dims=(), collapsed_slice_dims=(0,),
                    start_index_map=(0,))
                ss = ssv
                for s in (1, 2, 4, 8):
                    perm = (lane ^ s)[:, None]
                    ss = ss + lax.gather(
                        ss, perm, dn, slice_sizes=(1,),
                        mode=lax.GatherScatterMode.PROMISE_IN_BOUNDS)
                ib = lax.bitcast_convert_type(ss, jnp.int32)
                ih2 = jnp.int32(0x5F3759DF) - lax.shift_right_arithmetic(ib, 1)
                y = lax.bitcast_convert_type(ih2, jnp.float32)
                half = ss * 0.5
                for _ in range(3):
                    y = y * (1.5 - half * y * y)

                def nm_body(j, c, b=b, y=y):
                    sl = pl.ds(pl.multiple_of(j * 16, 16), 16)
                    acc_v[b, sl] = acc_v[b, sl] * y
                    return c

                lax.fori_loop(0, _H // 16, nm_body, 0, unroll=8)

            if m == acc_v.shape[0]:
                src_acc = acc_v
            else:
                src_acc = acc_v.at[pl.ds(0, m)]
            pltpu.sync_copy(src_acc,
                            out_hbm.at[pl.ds(wid * n_b + grp * m, m)])
            return carry

        lax.fori_loop(0, n_grp, grp_body, 0)

    return k(xg, flat_tab)


def _tc_body(x_ref, tab_ref, o_ref):
    x = x_ref[...]
    Bt, D = x.shape
    H = tab_ref.shape[-1]
    xn = jnp.clip(x * float(_L - 1), 0.0, float(_L - 1))
    low_f = jnp.floor(xn)
    a = xn - low_f
    low_i = low_f.astype(jnp.int32)
    high_i = jnp.minimum(low_i + 1, _L - 1)
    lvl = jax.lax.broadcasted_iota(jnp.int32, (Bt, _L), 1)
    acc = jnp.zeros((Bt, H), jnp.float32)
    for d in range(D):
        w = jnp.where(lvl == low_i[:, d:d + 1], 1.0 - a[:, d:d + 1], 0.0)
        w = w + jnp.where(lvl == high_i[:, d:d + 1], a[:, d:d + 1], 0.0)
        acc = acc + jnp.dot(w, tab_ref[d], preferred_element_type=jnp.float32)
    ss = jnp.sum(acc * acc, axis=1, keepdims=True)
    o_ref[...] = acc / jnp.sqrt(ss)


def _tc_encode(x_tc, base_hvs, bt, n_rows, out_rows):
    B, D = x_tc.shape
    _, L, H = base_hvs.shape
    return pl.pallas_call(
        _tc_body,
        grid=(n_rows // bt,),
        in_specs=[
            pl.BlockSpec((bt, D), lambda i: (i, 0)),
            pl.BlockSpec((D, L, H), lambda i: (0, 0, 0)),
        ],
        out_specs=pl.BlockSpec((bt, H), lambda i: (i, 0)),
        out_shape=jax.ShapeDtypeStruct((out_rows, H), jnp.float32),
    )(x_tc, base_hvs)


_N_SC = 64


def kernel(x, base_hvs):
    if x.ndim == 1:
        x = x[None, :]
    B, D = x.shape
    _, L, H = base_hvs.shape
    n_sc = _N_SC if B % 512 == 0 else 0
    if n_sc == 0:
        bt = 128 if B % 128 == 0 else B
        return _tc_encode(x, base_hvs, bt, B, B)
    n_tc = B - n_sc
    flat = base_hvs.reshape(D * L, H)
    out_tc = _tc_encode(x, base_hvs, 240, n_tc, B)
    out_sc = _sc_encode(x[n_tc:], flat, n_sc)
    return lax.dynamic_update_slice(out_tc, out_sc, (n_tc, 0))


# speedup vs baseline: 10.2553x; 1.1028x over previous
import functools

import jax
import jax.numpy as jnp
from jax import lax
from jax.experimental import pallas as pl
from jax.experimental.pallas import tpu as pltpu
from jax.experimental.pallas import tpu_sc as plsc

_D = 26
_L = 100
_H = 2048


def _sc_encode(x_sc, flat_tab, n_sc):
    info = plsc.get_sparse_core_info()
    nc, ns = info.num_cores, info.num_subcores
    nw = nc * ns
    n_b = n_sc // nw
    m = min(2, n_b)
    kf = 16 // (2 * m)
    dpad = -(-_D // kf) * kf
    nt = dpad // kf
    n_grp = n_b // m

    xp = jnp.pad(x_sc, ((0, 0), (0, dpad - _D)))
    xp = xp.reshape(nw, n_grp, m, nt, kf).transpose(0, 1, 3, 2, 4)
    xp = jnp.broadcast_to(xp[:, :, :, :, None, :],
                          (nw, n_grp, nt, m, 2, kf))
    xg = xp.reshape(nw * n_grp, nt * 16)

    mesh = plsc.VectorSubcoreMesh(core_axis_name="c", subcore_axis_name="s")

    @functools.partial(
        pl.kernel,
        mesh=mesh,
        out_type=jax.ShapeDtypeStruct((n_sc, _H), jnp.float32),
        scratch_types=[
            pltpu.VMEM((nt * 16,), jnp.float32),
            pltpu.VMEM((16, _H), jnp.float32),
            pltpu.VMEM((16, _H), jnp.float32),
            pltpu.VMEM((max(m, 2), _H), jnp.float32),
            pltpu.SemaphoreType.DMA,
            pltpu.SemaphoreType.DMA,
        ],
    )
    def k(xt_hbm, tab_hbm, out_hbm, xt_v, rows_a, rows_b, acc_v, sem_a, sem_b):
        wid = lax.axis_index("s") * nc + lax.axis_index("c")
        lane = lax.broadcasted_iota(jnp.int32, (16,), 0)
        lane_j = lane & (kf - 1)
        lane_s = lax.shift_right_logical(lane, kf.bit_length() - 1) & 1
        zero16f = jnp.zeros((16,), jnp.float32)
        bufs = [(rows_a, sem_a), (rows_b, sem_b)]

        def grp_body(grp, carry):
            pltpu.sync_copy(xt_hbm.at[wid * n_grp + grp], xt_v)

            ils, wss = [], []
            for t in range(nt):
                xv = xt_v[pl.ds(t * 16, 16)]
                xn = jnp.clip(xv * float(_L - 1), 0.0, float(_L - 1))
                low_i = xn.astype(jnp.int32)
                a = xn - low_i.astype(jnp.float32)
                high_i = jnp.minimum(low_i + 1, _L - 1)
                featv = lane_j + t * kf
                valid = featv < _D
                lvl = jnp.where(lane_s == 0, low_i, high_i)
                w = jnp.where(lane_s == 0, 1.0 - a, a)
                ils.append(jnp.where(valid, lvl + featv * _L, 0))
                wss.append(jnp.where(valid, w, zero16f))

            def gather(t):
                buf, sem = bufs[t % 2]
                return pltpu.async_copy(tab_hbm.at[ils[t]], buf, sem)

            gather(0)
            nr = 2 * kf
            for t in range(nt):
                if t + 1 < nt:
                    gather(t + 1)
                buf, sem = bufs[t % 2]
                pltpu.make_async_copy(tab_hbm.at[ils[t]], buf, sem).wait()
                for b in range(m):
                    ws = [wss[t][b * nr + r] for r in range(nr)]

                    def col_body(j, c, t=t, b=b, buf=buf, ws=ws):
                        sl = pl.ds(pl.multiple_of(j * 16, 16), 16)
                        v = ws[0] * buf[b * nr, sl]
                        for r in range(1, nr):
                            v = v + ws[r] * buf[b * nr + r, sl]
                        if t > 0:
                            v = v + acc_v[b, sl]
                        acc_v[b, sl] = v
                        return c

                    lax.fori_loop(0, _H // 16, col_body, 0,
                                  unroll=(64 // nr))

            for b in range(m):
                def ss_body(j, ssv, b=b):
                    v = acc_v[b, pl.ds(pl.multiple_of(j * 16, 16), 16)]
                    return ssv + v * v

                ssv = lax.fori_loop(0, _H // 16, ss_body, zero16f, unroll=8)
                dn = lax.GatherDimensionNumbers(
                    offset_dims=(), collapsed_slice_dims=(0,),
                    start_index_map=(0,))
                ss = ssv
                for s in (1, 2, 4, 8):
                    perm = (lane ^ s)[:, None]
                    ss = ss + lax.gather(
                        ss, perm, dn, slice_sizes=(1,),
                        mode=lax.GatherScatterMode.PROMISE_IN_BOUNDS)
                ib = lax.bitcast_convert_type(ss, jnp.int32)
                ih2 = jnp.int32(0x5F3759DF) - lax.shift_right_arithmetic(ib, 1)
                y = lax.bitcast_convert_type(ih2, jnp.float32)
                half = ss * 0.5
                for _ in range(3):
                    y = y * (1.5 - half * y * y)

                def nm_body(j, c, b=b, y=y):
                    sl = pl.ds(pl.multiple_of(j * 16, 16), 16)
                    acc_v[b, sl] = acc_v[b, sl] * y
                    return c

                lax.fori_loop(0, _H // 16, nm_body, 0, unroll=8)

            if m == acc_v.shape[0]:
                src_acc = acc_v
            else:
                src_acc = acc_v.at[pl.ds(0, m)]
            pltpu.sync_copy(src_acc,
                            out_hbm.at[pl.ds(wid * n_b + grp * m, m)])
            return carry

        lax.fori_loop(0, n_grp, grp_body, 0)

    return k(xg, flat_tab)


def _tc_body(x_ref, tab_ref, o_ref):
    x = x_ref[...]
    Bt, D = x.shape
    H = tab_ref.shape[-1]
    xn = jnp.clip(x * float(_L - 1), 0.0, float(_L - 1))
    low_f = jnp.floor(xn)
    a = xn - low_f
    low_i = low_f.astype(jnp.int32)
    high_i = jnp.minimum(low_i + 1, _L - 1)
    lvl = jax.lax.broadcasted_iota(jnp.int32, (Bt, _L), 1)
    acc = jnp.zeros((Bt, H), jnp.float32)
    for d in range(D):
        w = jnp.where(lvl == low_i[:, d:d + 1], 1.0 - a[:, d:d + 1], 0.0)
        w = w + jnp.where(lvl == high_i[:, d:d + 1], a[:, d:d + 1], 0.0)
        acc = acc + jnp.dot(w, tab_ref[d], preferred_element_type=jnp.float32)
    ss = jnp.sum(acc * acc, axis=1, keepdims=True)
    o_ref[...] = acc / jnp.sqrt(ss)


def _tc_encode(x_tc, base_hvs, bt, n_rows, out_rows):
    B, D = x_tc.shape
    _, L, H = base_hvs.shape
    return pl.pallas_call(
        _tc_body,
        grid=(n_rows // bt,),
        in_specs=[
            pl.BlockSpec((bt, D), lambda i: (i, 0)),
            pl.BlockSpec((D, L, H), lambda i: (0, 0, 0)),
        ],
        out_specs=pl.BlockSpec((bt, H), lambda i: (i, 0)),
        out_shape=jax.ShapeDtypeStruct((out_rows, H), jnp.float32),
    )(x_tc, base_hvs)


_N_SC = 32


def kernel(x, base_hvs):
    if x.ndim == 1:
        x = x[None, :]
    B, D = x.shape
    _, L, H = base_hvs.shape
    n_sc = _N_SC if B % 512 == 0 else 0
    if n_sc == 0:
        bt = 128 if B % 128 == 0 else B
        return _tc_encode(x, base_hvs, bt, B, B)
    n_tc = B - n_sc
    flat = base_hvs.reshape(D * L, H)
    out_tc = _tc_encode(x, base_hvs, 240, n_tc, B)
    out_sc = _sc_encode(x[n_tc:], flat, n_sc)
    return lax.dynamic_update_slice(out_tc, out_sc, (n_tc, 0))
